# Initial kernel scaffold; baseline (speedup 1.0000x reference)
import jax
import jax.numpy as jnp
from jax.experimental import pallas as pl


def kernel(sequence, lengths, W_enc, b_enc, W_proj, b_proj, trans, start, dur):
    raise NotImplementedError("write your pallas kernel here")


# speedup vs baseline: 9.6037x; 9.6037x over previous
import functools

import jax
import jax.numpy as jnp
from jax.experimental import pallas as pl
from jax.experimental.pallas import tpu as pltpu

_NEG = -1e30


def _encoder_kernel(seq_ref, wenc_ref, benc_ref, wproj_ref, bproj_ref,
                    out_ref, acc_ref):
    i = pl.program_id(1)
    x = seq_ref[0]
    h = jnp.dot(x, wenc_ref[...], preferred_element_type=jnp.float32)
    h = jax.nn.gelu(h + benc_ref[...])
    e = jnp.dot(h, wproj_ref[...], preferred_element_type=jnp.float32)
    e = e + bproj_ref[...]

    bt = e.shape[0]
    row = jax.lax.broadcasted_iota(jnp.int32, (bt, bt), 0)
    col = jax.lax.broadcasted_iota(jnp.int32, (bt, bt), 1)
    tril = (row >= col).astype(jnp.float32)
    cb = jnp.dot(tril, e, preferred_element_type=jnp.float32)

    @pl.when(i == 0)
    def _():
        acc_ref[...] = jnp.zeros_like(acc_ref)

    out = cb + acc_ref[...]
    out_ref[0] = out
    acc_ref[...] = out[bt - 1:bt, :]


def _dp_kernel(cumt_ref, durrev_ref, coeff_ref, startl_ref, len_ref,
               out_ref, *, T, K, C, B):
    L = C * B

    def group_lse(vec, add_coeff):
        v2 = jnp.concatenate([vec, vec], axis=1)
        terms = []
        for r in range(C):
            s = (C - r) * B
            t_r = v2[:, s:s + L]
            if add_coeff:
                t_r = t_r + coeff_ref[r:r + 1, :]
            terms.append(t_r)
        ts = jnp.concatenate(terms, axis=0)
        mx = jnp.max(ts, axis=0, keepdims=True)
        return mx + jnp.log(jnp.sum(jnp.exp(ts - mx), axis=0, keepdims=True))

    def body(t, carry):
        G, pacc = carry
        cur = cumt_ref[pl.ds(t + K, 1), :]
        score = G + durrev_ref[...]
        mx = jnp.max(score, axis=0, keepdims=True)
        alpha = cur + mx + jnp.log(
            jnp.sum(jnp.exp(score - mx), axis=0, keepdims=True))
        pacc = jnp.where(t == len_ref[...], alpha, pacc)
        m = group_lse(alpha, add_coeff=True)
        G = jnp.concatenate([G[1:], m - cur], axis=0)
        return (G, pacc)

    G0 = jnp.concatenate(
        [jnp.full((K - 1, L), _NEG, jnp.float32), startl_ref[...]], axis=0)
    _, pacc = jax.lax.fori_loop(
        1, T + 1, body, (G0, jnp.full((1, L), _NEG, jnp.float32)))
    out_ref[...] = group_lse(pacc, add_coeff=False)


def kernel(sequence, lengths, W_enc, b_enc, W_proj, b_proj, trans, start, dur):
    B, T, D = sequence.shape
    HID = W_enc.shape[1]
    C = W_proj.shape[1]
    K = dur.shape[1]
    L = C * B
    BT = 512
    NT = T // BT

    cum_body = pl.pallas_call(
        _encoder_kernel,
        grid=(B, NT),
        in_specs=[
            pl.BlockSpec((1, BT, D), lambda b, i: (b, i, 0)),
            pl.BlockSpec((D, HID), lambda b, i: (0, 0)),
            pl.BlockSpec((1, HID), lambda b, i: (0, 0)),
            pl.BlockSpec((HID, C), lambda b, i: (0, 0)),
            pl.BlockSpec((1, C), lambda b, i: (0, 0)),
        ],
        out_specs=pl.BlockSpec((1, BT, C), lambda b, i: (b, i, 0)),
        out_shape=jax.ShapeDtypeStruct((B, T, C), jnp.float32),
        scratch_shapes=[pltpu.VMEM((1, C), jnp.float32)],
        compiler_params=pltpu.CompilerParams(
            dimension_semantics=("parallel", "arbitrary")),
    )(sequence, W_enc, b_enc.reshape(1, HID), W_proj, b_proj.reshape(1, C))

    cum = jnp.concatenate(
        [jnp.zeros((B, 1, C), cum_body.dtype), cum_body], axis=1)

    cumt = cum.transpose(1, 2, 0).reshape(T + 1, L)
    cumt = jnp.concatenate([jnp.zeros((K, L), jnp.float32), cumt], axis=0)
    dur_rev = jnp.repeat(dur[:, ::-1].T, B, axis=1)
    cidx = jnp.arange(C)
    rows = (cidx[None, :] - cidx[:, None]) % C
    coeff = jnp.repeat(trans[rows, cidx[None, :]], B, axis=1)
    startl = jnp.repeat(start[:, None], B, axis=1).reshape(1, L)
    lenl = jnp.tile(lengths, C).reshape(1, L).astype(jnp.int32)

    part_l = pl.pallas_call(
        functools.partial(_dp_kernel, T=T, K=K, C=C, B=B),
        out_shape=jax.ShapeDtypeStruct((1, L), jnp.float32),
    )(cumt, dur_rev, coeff, startl, lenl)

    partition = part_l[0, 0:B]
    return partition, cum
